# baseline (speedup 1.0000x reference)
import functools

import jax
import jax.numpy as jnp
from jax import lax
from jax.experimental import pallas as pl
from jax.experimental.pallas import tpu as pltpu
from jax.experimental.pallas import tpu_sc as plsc

_NC, _NS, _L = 2, 16, 16
_NW = _NC * _NS
_N = 10000
_D = 128
_E = 320000
_B = 16
_EP = 327680
_CH = _EP // _NW
_RW = _CH // 128
_CHKR = 16
_NCHK = _RW // _CHKR
_NPAD = 10240
_RPT = _NPAD // _NS
_HR = 80
_HN = _HR * 128

_R = 1024
_G = _NPAD // _R


def _mesh():
    return plsc.VectorSubcoreMesh(core_axis_name="c", subcore_axis_name="s",
                                  num_cores=_NC, num_subcores=_NS)


@functools.partial(
    pl.kernel,
    out_type=jax.ShapeDtypeStruct((_NW * _HN,), jnp.float32),
    mesh=_mesh(),
    compiler_params=pltpu.CompilerParams(needs_layout_passes=False),
    scratch_types=[
        pltpu.VMEM((_CH,), jnp.int32),
        pltpu.VMEM((_HN,), jnp.float32),
    ])
def _deg_kernel(dst_hbm, out_hbm, dst_v, hist_v):
    cid = lax.axis_index("c")
    sid = lax.axis_index("s")
    w = cid * _NS + sid
    zeros = jnp.zeros((_L,), jnp.float32)
    ones = jnp.ones((_L,), jnp.float32)

    def zbody(i, carry):
        hist_v[pl.ds(i * _L, _L)] = zeros
        return carry
    lax.fori_loop(0, _HN // _L, zbody, 0)

    pltpu.sync_copy(dst_hbm.at[pl.ds(w * _CH, _CH)], dst_v)

    def body(i, carry):
        v = dst_v[pl.ds(i * _L, _L)]
        plsc.addupdate_scatter(hist_v, [v], ones)
        return carry
    lax.fori_loop(0, _CH // _L, body, 0)

    pltpu.sync_copy(hist_v, out_hbm.at[pl.ds(w * _HN, _HN)])


@functools.partial(
    pl.kernel,
    out_type=jax.ShapeDtypeStruct((_NC, _NPAD, _D), jnp.float32),
    mesh=_mesh(),
    compiler_params=pltpu.CompilerParams(needs_layout_passes=False),
    scratch_types=[
        [pltpu.VMEM((_CHKR, 128), jnp.int32)] * 2,
        [pltpu.VMEM((2 * _CHKR, 64), jnp.int32)] * 2,
        [pltpu.VMEM((128, _D), jnp.float32)] * 2,
        [pltpu.SemaphoreType.DMA] * 2,
        [pltpu.SemaphoreType.DMA] * 4,
        [pltpu.SemaphoreType.DMA] * 2,
        [pltpu.SemaphoreType.DMA] * 2,
        pltpu.VMEM_SHARED((_NPAD, _D), jnp.float32),
    ])
def _agg_kernel(g_hbm, src_hbm, dst_hbm, out_hbm,
                src_c, dst_c, gbufs, gsems, scsems, ssems, dsems, acc_sh):
    cid = lax.axis_index("c")
    sid = lax.axis_index("s")
    w = cid * _NS + sid
    base = w * _RW

    def load_chunk(c):
        p = c % 2
        off = pl.multiple_of(base + c * _CHKR, 8)
        hoff = pl.multiple_of((base + c * _CHKR) * 2, 8)
        pltpu.async_copy(src_hbm.at[pl.ds(off, _CHKR)], src_c[p], ssems[p])
        pltpu.async_copy(dst_hbm.at[pl.ds(hoff, 2 * _CHKR)], dst_c[p], dsems[p])

    def wait_chunk(c):
        p = c % 2
        off = pl.multiple_of(base + c * _CHKR, 8)
        hoff = pl.multiple_of((base + c * _CHKR) * 2, 8)
        pltpu.make_async_copy(src_hbm.at[pl.ds(off, _CHKR)], src_c[p], ssems[p]).wait()
        pltpu.make_async_copy(dst_hbm.at[pl.ds(hoff, 2 * _CHKR)], dst_c[p], dsems[p]).wait()

    def issue_gather(p, row, b):
        pltpu.async_copy(g_hbm.at[src_c[p].at[row]], gbufs[b], gsems[b])

    def wait_gather(p, row, b):
        pltpu.make_async_copy(g_hbm.at[src_c[p].at[row]], gbufs[b], gsems[b]).wait()

    def scatter(p, row, b):
        for h in range(2):
            pltpu.async_copy(gbufs[b].at[pl.ds(h * 64, 64)],
                             acc_sh.at[dst_c[p].at[2 * row + h]],
                             scsems[2 * b + h], add=True)
        for h in range(2):
            pltpu.make_async_copy(gbufs[b].at[pl.ds(h * 64, 64)],
                                  acc_sh.at[dst_c[p].at[2 * row + h]],
                                  scsems[2 * b + h]).wait()

    ioff = pl.multiple_of(sid * _RPT, 8)
    pltpu.sync_copy(g_hbm.at[pl.ds(ioff, _RPT)], acc_sh.at[pl.ds(ioff, _RPT)])

    load_chunk(0)
    wait_chunk(0)
    load_chunk(1)
    plsc.subcore_barrier()

    for b in range(2):
        issue_gather(0, b, b)

    for c in range(_NCHK):
        p = c % 2

        def grp_body(grp, carry, p=p):
            for b in range(2):
                j = grp * 2 + b
                wait_gather(p, j, b)
                scatter(p, j, b)
                issue_gather(p, j + 2, b)
            return carry
        lax.fori_loop(0, _CHKR // 2 - 1, grp_body, 0)

        for b in range(2):
            j = _CHKR - 2 + b
            wait_gather(p, j, b)
            scatter(p, j, b)

        if c < _NCHK - 1:
            pn = (c + 1) % 2
            wait_chunk(c + 1)
            if c + 2 < _NCHK:
                load_chunk(c + 2)
            for b in range(2):
                issue_gather(pn, b, b)

    plsc.subcore_barrier()
    pltpu.sync_copy(acc_sh.at[pl.ds(ioff, _RPT)],
                    out_hbm.at[cid, pl.ds(ioff, _RPT)])


def _tc1_body(x_ref, w_ref, degp_ref, out_ref, dis_ref):
    ones = jnp.ones((_NW, 1), jnp.float32)
    deg = lax.dot_general(degp_ref[...], ones, (((0,), (0,)), ((), ())),
                          preferred_element_type=jnp.float32) + 1.0
    dis = lax.rsqrt(deg)
    dis_ref[...] = jnp.broadcast_to(dis, (_R, _D))
    out_ref[...] = jnp.dot(x_ref[...], w_ref[...],
                           preferred_element_type=jnp.float32) * dis


def _tc1(x, w1, deg_parts):
    return pl.pallas_call(
        _tc1_body,
        grid=(_G,),
        in_specs=[
            pl.BlockSpec((_R, _D), lambda i: (i, 0)),
            pl.BlockSpec((_D, _D), lambda i: (0, 0)),
            pl.BlockSpec((_NW, _R), lambda i: (0, i)),
        ],
        out_specs=[
            pl.BlockSpec((_R, _D), lambda i: (i, 0)),
            pl.BlockSpec((_R, _D), lambda i: (i, 0)),
        ],
        out_shape=[
            jax.ShapeDtypeStruct((_NPAD, _D), jnp.float32),
            jax.ShapeDtypeStruct((_NPAD, _D), jnp.float32),
        ],
    )(x, w1, deg_parts)


def _tc2_body(acc_ref, g_ref, dis_ref, b_ref, w_ref, out_ref):
    dis = dis_ref[...]
    a = acc_ref[0] + acc_ref[1] - g_ref[...]
    h = jnp.maximum(a * dis + b_ref[...], 0.0)
    out_ref[...] = jnp.dot(h, w_ref[...],
                           preferred_element_type=jnp.float32) * dis


def _tc2(acc, g, dis_b, b1r, w2):
    return pl.pallas_call(
        _tc2_body,
        grid=(_G,),
        in_specs=[
            pl.BlockSpec((_NC, _R, _D), lambda i: (0, i, 0)),
            pl.BlockSpec((_R, _D), lambda i: (i, 0)),
            pl.BlockSpec((_R, _D), lambda i: (i, 0)),
            pl.BlockSpec((1, _D), lambda i: (0, 0)),
            pl.BlockSpec((_D, _D), lambda i: (0, 0)),
        ],
        out_specs=pl.BlockSpec((_R, _D), lambda i: (i, 0)),
        out_shape=jax.ShapeDtypeStruct((_NPAD, _D), jnp.float32),
    )(acc, g, dis_b, b1r, w2)


def _tc3_body(acc_ref, g_ref, dis_ref, b_ref, batch_ref, wfc_ref, bfc_ref,
              out_ref, psum, cnt):
    i = pl.program_id(0)

    @pl.when(i == 0)
    def _():
        psum[...] = jnp.zeros_like(psum)
        cnt[...] = jnp.zeros_like(cnt)

    dis = dis_ref[...]
    a = acc_ref[0] + acc_ref[1] - g_ref[...]
    h = jnp.maximum(a * dis + b_ref[...], 0.0)
    ids = lax.broadcasted_iota(jnp.int32, (_B, _R), 0)
    mask = (ids == batch_ref[...].reshape(1, _R)).astype(jnp.float32)
    psum[...] += jnp.dot(mask, h, preferred_element_type=jnp.float32)
    cnt[...] += jnp.broadcast_to(jnp.sum(mask, axis=1, keepdims=True), (_B, _D))

    @pl.when(i == _G - 1)
    def _():
        pooled = psum[...] / jnp.maximum(cnt[...], 1.0)
        out_ref[...] = jnp.dot(pooled, wfc_ref[...],
                               preferred_element_type=jnp.float32) + bfc_ref[...]


def _tc3(acc, g, dis_b, b2r, batch3, wfc, bfcr):
    fco = wfc.shape[1]
    return pl.pallas_call(
        _tc3_body,
        grid=(_G,),
        in_specs=[
            pl.BlockSpec((_NC, _R, _D), lambda i: (0, i, 0)),
            pl.BlockSpec((_R, _D), lambda i: (i, 0)),
            pl.BlockSpec((_R, _D), lambda i: (i, 0)),
            pl.BlockSpec((1, _D), lambda i: (0, 0)),
            pl.BlockSpec((1, 1, _R), lambda i: (i, 0, 0)),
            pl.BlockSpec((_D, fco), lambda i: (0, 0)),
            pl.BlockSpec((1, fco), lambda i: (0, 0)),
        ],
        out_specs=pl.BlockSpec((_B, fco), lambda i: (0, 0)),
        out_shape=jax.ShapeDtypeStruct((_B, fco), jnp.float32),
        scratch_shapes=[
            pltpu.VMEM((_B, _D), jnp.float32),
            pltpu.VMEM((_B, _D), jnp.float32),
        ],
    )(acc, g, dis_b, b2r, batch3, wfc, bfcr)


def kernel(x, edge_index, batch, W1, b1, W2, b2, Wfc, bfc):
    pad = _EP - _E
    lanes = jnp.arange(pad, dtype=jnp.int32) % 128
    src = jnp.concatenate([edge_index[0], lanes])
    dst = jnp.concatenate([edge_index[1], _N + lanes])
    src2 = src.reshape(_EP // 128, 128)
    dst2 = dst.reshape(_EP // 64, 64)

    deg_parts = _deg_kernel(dst).reshape(_NW, _HN)
    xp = jnp.pad(x, ((0, _NPAD - _N), (0, 0)))
    batchp = jnp.pad(batch, (0, _NPAD - _N), constant_values=_B)

    g1, dis_b = _tc1(xp, W1, deg_parts)
    acc1 = _agg_kernel(g1, src2, dst2)
    g2 = _tc2(acc1, g1, dis_b, b1.reshape(1, _D), W2)
    acc2 = _agg_kernel(g2, src2, dst2)
    out = _tc3(acc2, g2, dis_b, b2.reshape(1, _D), batchp.reshape(_G, 1, _R),
               Wfc, bfc.reshape(1, -1))
    return out.reshape(_B, 4, 32, 64)


# speedup vs baseline: 1.0100x; 1.0100x over previous
import functools

import jax
import jax.numpy as jnp
from jax import lax
from jax.experimental import pallas as pl
from jax.experimental.pallas import tpu as pltpu
from jax.experimental.pallas import tpu_sc as plsc

_NC, _NS, _L = 2, 16, 16
_NW = _NC * _NS
_N = 10000
_D = 128
_E = 320000
_B = 16
_EP = 327680
_CH = _EP // _NW
_RW = _CH // 128
_CHKR = 16
_NCHK = _RW // _CHKR
_NPAD = 10240
_RPT = _NPAD // _NS
_HR = 80
_HN = _HR * 128

_R = 1024
_G = _NPAD // _R


def _mesh():
    return plsc.VectorSubcoreMesh(core_axis_name="c", subcore_axis_name="s",
                                  num_cores=_NC, num_subcores=_NS)


@functools.partial(
    pl.kernel,
    out_type=jax.ShapeDtypeStruct((_NW * _HN,), jnp.float32),
    mesh=_mesh(),
    compiler_params=pltpu.CompilerParams(needs_layout_passes=False),
    scratch_types=[
        pltpu.VMEM((_CH,), jnp.int32),
        pltpu.VMEM((_HN,), jnp.float32),
    ])
def _deg_kernel(dst_hbm, out_hbm, dst_v, hist_v):
    cid = lax.axis_index("c")
    sid = lax.axis_index("s")
    w = cid * _NS + sid
    zeros = jnp.zeros((_L,), jnp.float32)
    ones = jnp.ones((_L,), jnp.float32)

    def zbody(i, carry):
        hist_v[pl.ds(i * _L, _L)] = zeros
        return carry
    lax.fori_loop(0, _HN // _L, zbody, 0)

    pltpu.sync_copy(dst_hbm.at[pl.ds(w * _CH, _CH)], dst_v)

    def body(i, carry):
        v = dst_v[pl.ds(i * _L, _L)]
        plsc.addupdate_scatter(hist_v, [v], ones)
        return carry
    lax.fori_loop(0, _CH // _L, body, 0)

    pltpu.sync_copy(hist_v, out_hbm.at[pl.ds(w * _HN, _HN)])


@functools.partial(
    pl.kernel,
    out_type=jax.ShapeDtypeStruct((_NC, _NPAD, _D), jnp.float32),
    mesh=_mesh(),
    compiler_params=pltpu.CompilerParams(needs_layout_passes=False),
    scratch_types=[
        [pltpu.VMEM((_CHKR, 128), jnp.int32)] * 2,
        [pltpu.VMEM((_CHKR, 128), jnp.int32)] * 2,
        [pltpu.VMEM((128, _D), jnp.float32)] * 2,
        [pltpu.SemaphoreType.DMA] * 2,
        [pltpu.SemaphoreType.DMA] * 2,
        [pltpu.SemaphoreType.DMA] * 2,
        pltpu.VMEM_SHARED((_NPAD, _D), jnp.float32),
    ])
def _agg_kernel(g_hbm, src_hbm, dst_hbm, out_hbm,
                src_c, dst_c, gbufs, gsems, ssems, dsems, acc_sh):
    cid = lax.axis_index("c")
    sid = lax.axis_index("s")
    w = cid * _NS + sid
    base = w * _RW

    def load_chunk(c):
        p = c % 2
        off = pl.multiple_of(base + c * _CHKR, 8)
        pltpu.async_copy(src_hbm.at[pl.ds(off, _CHKR)], src_c[p], ssems[p])
        pltpu.async_copy(dst_hbm.at[pl.ds(off, _CHKR)], dst_c[p], dsems[p])

    def wait_chunk(c):
        p = c % 2
        off = pl.multiple_of(base + c * _CHKR, 8)
        pltpu.make_async_copy(src_hbm.at[pl.ds(off, _CHKR)], src_c[p], ssems[p]).wait()
        pltpu.make_async_copy(dst_hbm.at[pl.ds(off, _CHKR)], dst_c[p], dsems[p]).wait()

    def issue_gather(p, row, b):
        pltpu.async_copy(g_hbm.at[src_c[p].at[row]], gbufs[b], gsems[b])

    def wait_gather(p, row, b):
        pltpu.make_async_copy(g_hbm.at[src_c[p].at[row]], gbufs[b], gsems[b]).wait()

    def scatter(p, row, b):
        pltpu.sync_copy(gbufs[b], acc_sh.at[dst_c[p].at[row]], add=True)

    ioff = pl.multiple_of(sid * _RPT, 8)
    pltpu.sync_copy(g_hbm.at[pl.ds(ioff, _RPT)], acc_sh.at[pl.ds(ioff, _RPT)])

    load_chunk(0)
    wait_chunk(0)
    load_chunk(1)
    plsc.subcore_barrier()

    for b in range(2):
        issue_gather(0, b, b)

    for c in range(_NCHK):
        p = c % 2

        def grp_body(grp, carry, p=p):
            for b in range(2):
                j = grp * 2 + b
                wait_gather(p, j, b)
                scatter(p, j, b)
                issue_gather(p, j + 2, b)
            return carry
        lax.fori_loop(0, _CHKR // 2 - 1, grp_body, 0)

        for b in range(2):
            j = _CHKR - 2 + b
            wait_gather(p, j, b)
            scatter(p, j, b)

        if c < _NCHK - 1:
            pn = (c + 1) % 2
            wait_chunk(c + 1)
            if c + 2 < _NCHK:
                load_chunk(c + 2)
            for b in range(2):
                issue_gather(pn, b, b)

    plsc.subcore_barrier()
    pltpu.sync_copy(acc_sh.at[pl.ds(ioff, _RPT)],
                    out_hbm.at[cid, pl.ds(ioff, _RPT)])


def _dis_col(degp_ref):
    ones = jnp.ones((_NW, 1), jnp.float32)
    deg = lax.dot_general(degp_ref[...], ones, (((0,), (0,)), ((), ())),
                          preferred_element_type=jnp.float32) + 1.0
    return lax.rsqrt(deg)


def _tc1_body(x_ref, w_ref, degp_ref, out_ref):
    dis = _dis_col(degp_ref)
    rowid = lax.broadcasted_iota(jnp.int32, (_R, 1), 0) + pl.program_id(0) * _R
    dis = jnp.where(rowid < _N, dis, 0.0)
    out_ref[...] = jnp.dot(x_ref[...], w_ref[...],
                           preferred_element_type=jnp.float32) * dis


def _tc1(x, w1, deg_parts):
    return pl.pallas_call(
        _tc1_body,
        grid=(_G,),
        in_specs=[
            pl.BlockSpec((_R, _D), lambda i: (i, 0)),
            pl.BlockSpec((_D, _D), lambda i: (0, 0)),
            pl.BlockSpec((_NW, _R), lambda i: (0, i)),
        ],
        out_specs=pl.BlockSpec((_R, _D), lambda i: (i, 0)),
        out_shape=jax.ShapeDtypeStruct((_NPAD, _D), jnp.float32),
    )(x, w1, deg_parts)


def _tc2_body(acc_ref, g_ref, degp_ref, b_ref, w_ref, out_ref):
    dis = _dis_col(degp_ref)
    a = acc_ref[0] + acc_ref[1] - g_ref[...]
    h = jnp.maximum(a * dis + b_ref[...], 0.0)
    out_ref[...] = jnp.dot(h, w_ref[...],
                           preferred_element_type=jnp.float32) * dis


def _tc2(acc, g, deg_parts, b1r, w2):
    return pl.pallas_call(
        _tc2_body,
        grid=(_G,),
        in_specs=[
            pl.BlockSpec((_NC, _R, _D), lambda i: (0, i, 0)),
            pl.BlockSpec((_R, _D), lambda i: (i, 0)),
            pl.BlockSpec((_NW, _R), lambda i: (0, i)),
            pl.BlockSpec((1, _D), lambda i: (0, 0)),
            pl.BlockSpec((_D, _D), lambda i: (0, 0)),
        ],
        out_specs=pl.BlockSpec((_R, _D), lambda i: (i, 0)),
        out_shape=jax.ShapeDtypeStruct((_NPAD, _D), jnp.float32),
    )(acc, g, deg_parts, b1r, w2)


def _tc3_body(acc_ref, g_ref, degp_ref, b_ref, batch_ref, wfc_ref, bfc_ref,
              out_ref, psum, cnt):
    i = pl.program_id(0)

    @pl.when(i == 0)
    def _():
        psum[...] = jnp.zeros_like(psum)
        cnt[...] = jnp.zeros_like(cnt)

    dis = _dis_col(degp_ref)
    a = acc_ref[0] + acc_ref[1] - g_ref[...]
    h = jnp.maximum(a * dis + b_ref[...], 0.0)
    ids = lax.broadcasted_iota(jnp.int32, (_B, _R), 0)
    mask = (ids == batch_ref[...].reshape(1, _R)).astype(jnp.float32)
    psum[...] += jnp.dot(mask, h, preferred_element_type=jnp.float32)
    cnt[...] += jnp.broadcast_to(jnp.sum(mask, axis=1, keepdims=True), (_B, _D))

    @pl.when(i == _G - 1)
    def _():
        pooled = psum[...] / jnp.maximum(cnt[...], 1.0)
        out_ref[...] = jnp.dot(pooled, wfc_ref[...],
                               preferred_element_type=jnp.float32) + bfc_ref[...]


def _tc3(acc, g, deg_parts, b2r, batch3, wfc, bfcr):
    fco = wfc.shape[1]
    return pl.pallas_call(
        _tc3_body,
        grid=(_G,),
        in_specs=[
            pl.BlockSpec((_NC, _R, _D), lambda i: (0, i, 0)),
            pl.BlockSpec((_R, _D), lambda i: (i, 0)),
            pl.BlockSpec((_NW, _R), lambda i: (0, i)),
            pl.BlockSpec((1, _D), lambda i: (0, 0)),
            pl.BlockSpec((1, 1, _R), lambda i: (i, 0, 0)),
            pl.BlockSpec((_D, fco), lambda i: (0, 0)),
            pl.BlockSpec((1, fco), lambda i: (0, 0)),
        ],
        out_specs=pl.BlockSpec((_B, fco), lambda i: (0, 0)),
        out_shape=jax.ShapeDtypeStruct((_B, fco), jnp.float32),
        scratch_shapes=[
            pltpu.VMEM((_B, _D), jnp.float32),
            pltpu.VMEM((_B, _D), jnp.float32),
        ],
    )(acc, g, deg_parts, b2r, batch3, wfc, bfcr)


def kernel(x, edge_index, batch, W1, b1, W2, b2, Wfc, bfc):
    pad = _EP - _E
    lanes = jnp.arange(pad, dtype=jnp.int32) % 128
    src = jnp.concatenate([edge_index[0], lanes])
    dst = jnp.concatenate([edge_index[1], _N + lanes])
    src2 = src.reshape(_EP // 128, 128)
    dst2 = dst.reshape(_EP // 128, 128)

    deg_parts = _deg_kernel(dst).reshape(_NW, _HN)
    xp = jnp.pad(x, ((0, _NPAD - _N), (0, 0)))
    batchp = jnp.pad(batch, (0, _NPAD - _N), constant_values=_B)

    g1 = _tc1(xp, W1, deg_parts)
    acc1 = _agg_kernel(g1, src2, dst2)
    g2 = _tc2(acc1, g1, deg_parts, b1.reshape(1, _D), W2)
    acc2 = _agg_kernel(g2, src2, dst2)
    out = _tc3(acc2, g2, deg_parts, b2.reshape(1, _D), batchp.reshape(_G, 1, _R),
               Wfc, bfc.reshape(1, -1))
    return out.reshape(_B, 4, 32, 64)


# speedup vs baseline: 1.0329x; 1.0226x over previous
import functools

import jax
import jax.numpy as jnp
from jax import lax
from jax.experimental import pallas as pl
from jax.experimental.pallas import tpu as pltpu
from jax.experimental.pallas import tpu_sc as plsc

_NC, _NS, _L = 2, 16, 16
_NW = _NC * _NS
_N = 10000
_D = 128
_E = 320000
_B = 16
_EP = 327680
_CH = _EP // _NW
_RW = _CH // 128
_CHKR = 16
_NCHK = _RW // _CHKR
_NPAD = 10240
_RPT = _NPAD // _NS
_HR = 80
_HN = _HR * 128

_R = 2048
_G = _NPAD // _R


def _mesh():
    return plsc.VectorSubcoreMesh(core_axis_name="c", subcore_axis_name="s",
                                  num_cores=_NC, num_subcores=_NS)


@functools.partial(
    pl.kernel,
    out_type=jax.ShapeDtypeStruct((_NW * _HN,), jnp.float32),
    mesh=_mesh(),
    compiler_params=pltpu.CompilerParams(needs_layout_passes=False),
    scratch_types=[
        pltpu.VMEM((_CH,), jnp.int32),
        pltpu.VMEM((_HN,), jnp.float32),
    ])
def _deg_kernel(dst_hbm, out_hbm, dst_v, hist_v):
    cid = lax.axis_index("c")
    sid = lax.axis_index("s")
    w = cid * _NS + sid
    zeros = jnp.zeros((_L,), jnp.float32)
    ones = jnp.ones((_L,), jnp.float32)

    def zbody(i, carry):
        hist_v[pl.ds(i * _L, _L)] = zeros
        return carry
    lax.fori_loop(0, _HN // _L, zbody, 0)

    pltpu.sync_copy(dst_hbm.at[pl.ds(w * _CH, _CH)], dst_v)

    def body(i, carry):
        v = dst_v[pl.ds(i * _L, _L)]
        plsc.addupdate_scatter(hist_v, [v], ones)
        return carry
    lax.fori_loop(0, _CH // _L, body, 0)

    pltpu.sync_copy(hist_v, out_hbm.at[pl.ds(w * _HN, _HN)])


@functools.partial(
    pl.kernel,
    out_type=jax.ShapeDtypeStruct((_NC, _NPAD, _D), jnp.float32),
    mesh=_mesh(),
    compiler_params=pltpu.CompilerParams(needs_layout_passes=False),
    scratch_types=[
        [pltpu.VMEM((_CHKR, 128), jnp.int32)] * 2,
        [pltpu.VMEM((_CHKR, 128), jnp.int32)] * 2,
        [pltpu.VMEM((128, _D), jnp.float32)] * 2,
        [pltpu.SemaphoreType.DMA] * 2,
        [pltpu.SemaphoreType.DMA] * 2,
        [pltpu.SemaphoreType.DMA] * 2,
        pltpu.VMEM_SHARED((_NPAD, _D), jnp.float32),
    ])
def _agg_kernel(g_hbm, src_hbm, dst_hbm, out_hbm,
                src_c, dst_c, gbufs, gsems, ssems, dsems, acc_sh):
    cid = lax.axis_index("c")
    sid = lax.axis_index("s")
    w = cid * _NS + sid
    base = w * _RW

    def load_chunk(c):
        p = c % 2
        off = pl.multiple_of(base + c * _CHKR, 8)
        pltpu.async_copy(src_hbm.at[pl.ds(off, _CHKR)], src_c[p], ssems[p])
        pltpu.async_copy(dst_hbm.at[pl.ds(off, _CHKR)], dst_c[p], dsems[p])

    def wait_chunk(c):
        p = c % 2
        off = pl.multiple_of(base + c * _CHKR, 8)
        pltpu.make_async_copy(src_hbm.at[pl.ds(off, _CHKR)], src_c[p], ssems[p]).wait()
        pltpu.make_async_copy(dst_hbm.at[pl.ds(off, _CHKR)], dst_c[p], dsems[p]).wait()

    def issue_gather(p, row, b):
        pltpu.async_copy(g_hbm.at[src_c[p].at[row]], gbufs[b], gsems[b])

    def wait_gather(p, row, b):
        pltpu.make_async_copy(g_hbm.at[src_c[p].at[row]], gbufs[b], gsems[b]).wait()

    def scatter(p, row, b):
        pltpu.sync_copy(gbufs[b], acc_sh.at[dst_c[p].at[row]], add=True)

    ioff = pl.multiple_of(sid * _RPT, 8)
    pltpu.sync_copy(g_hbm.at[pl.ds(ioff, _RPT)], acc_sh.at[pl.ds(ioff, _RPT)])

    load_chunk(0)
    wait_chunk(0)
    load_chunk(1)
    plsc.subcore_barrier()

    for b in range(2):
        issue_gather(0, b, b)

    for c in range(_NCHK):
        p = c % 2

        def grp_body(grp, carry, p=p):
            for b in range(2):
                j = grp * 2 + b
                wait_gather(p, j, b)
                scatter(p, j, b)
                issue_gather(p, j + 2, b)
            return carry
        lax.fori_loop(0, _CHKR // 2 - 1, grp_body, 0)

        for b in range(2):
            j = _CHKR - 2 + b
            wait_gather(p, j, b)
            scatter(p, j, b)

        if c < _NCHK - 1:
            pn = (c + 1) % 2
            wait_chunk(c + 1)
            if c + 2 < _NCHK:
                load_chunk(c + 2)
            for b in range(2):
                issue_gather(pn, b, b)

    plsc.subcore_barrier()
    pltpu.sync_copy(acc_sh.at[pl.ds(ioff, _RPT)],
                    out_hbm.at[cid, pl.ds(ioff, _RPT)])


def _dis_col(degp_ref):
    ones = jnp.ones((_NW, 1), jnp.float32)
    deg = lax.dot_general(degp_ref[...], ones, (((0,), (0,)), ((), ())),
                          preferred_element_type=jnp.float32) + 1.0
    return lax.rsqrt(deg)


def _tc1_body(x_ref, w_ref, degp_ref, out_ref):
    dis = _dis_col(degp_ref)
    rowid = lax.broadcasted_iota(jnp.int32, (_R, 1), 0) + pl.program_id(0) * _R
    dis = jnp.where(rowid < _N, dis, 0.0)
    out_ref[...] = jnp.dot(x_ref[...], w_ref[...],
                           preferred_element_type=jnp.float32) * dis


def _tc1(x, w1, deg_parts):
    return pl.pallas_call(
        _tc1_body,
        grid=(_G,),
        in_specs=[
            pl.BlockSpec((_R, _D), lambda i: (i, 0)),
            pl.BlockSpec((_D, _D), lambda i: (0, 0)),
            pl.BlockSpec((_NW, _R), lambda i: (0, i)),
        ],
        out_specs=pl.BlockSpec((_R, _D), lambda i: (i, 0)),
        out_shape=jax.ShapeDtypeStruct((_NPAD, _D), jnp.float32),
    )(x, w1, deg_parts)


def _tc2_body(acc_ref, g_ref, degp_ref, b_ref, w_ref, out_ref):
    dis = _dis_col(degp_ref)
    a = acc_ref[0] + acc_ref[1] - g_ref[...]
    h = jnp.maximum(a * dis + b_ref[...], 0.0)
    out_ref[...] = jnp.dot(h, w_ref[...],
                           preferred_element_type=jnp.float32) * dis


def _tc2(acc, g, deg_parts, b1r, w2):
    return pl.pallas_call(
        _tc2_body,
        grid=(_G,),
        in_specs=[
            pl.BlockSpec((_NC, _R, _D), lambda i: (0, i, 0)),
            pl.BlockSpec((_R, _D), lambda i: (i, 0)),
            pl.BlockSpec((_NW, _R), lambda i: (0, i)),
            pl.BlockSpec((1, _D), lambda i: (0, 0)),
            pl.BlockSpec((_D, _D), lambda i: (0, 0)),
        ],
        out_specs=pl.BlockSpec((_R, _D), lambda i: (i, 0)),
        out_shape=jax.ShapeDtypeStruct((_NPAD, _D), jnp.float32),
    )(acc, g, deg_parts, b1r, w2)


def _tc3_body(acc_ref, g_ref, degp_ref, b_ref, batch_ref, wfc_ref, bfc_ref,
              out_ref, psum, cnt):
    i = pl.program_id(0)

    @pl.when(i == 0)
    def _():
        psum[...] = jnp.zeros_like(psum)
        cnt[...] = jnp.zeros_like(cnt)

    dis = _dis_col(degp_ref)
    a = acc_ref[0] + acc_ref[1] - g_ref[...]
    h = jnp.maximum(a * dis + b_ref[...], 0.0)
    ids = lax.broadcasted_iota(jnp.int32, (_B, _R), 0)
    mask = (ids == batch_ref[...].reshape(1, _R)).astype(jnp.float32)
    psum[...] += jnp.dot(mask, h, preferred_element_type=jnp.float32)
    cnt[...] += jnp.broadcast_to(jnp.sum(mask, axis=1, keepdims=True), (_B, _D))

    @pl.when(i == _G - 1)
    def _():
        pooled = psum[...] / jnp.maximum(cnt[...], 1.0)
        out_ref[...] = jnp.dot(pooled, wfc_ref[...],
                               preferred_element_type=jnp.float32) + bfc_ref[...]


def _tc3(acc, g, deg_parts, b2r, batch3, wfc, bfcr):
    fco = wfc.shape[1]
    return pl.pallas_call(
        _tc3_body,
        grid=(_G,),
        in_specs=[
            pl.BlockSpec((_NC, _R, _D), lambda i: (0, i, 0)),
            pl.BlockSpec((_R, _D), lambda i: (i, 0)),
            pl.BlockSpec((_NW, _R), lambda i: (0, i)),
            pl.BlockSpec((1, _D), lambda i: (0, 0)),
            pl.BlockSpec((1, 1, _R), lambda i: (i, 0, 0)),
            pl.BlockSpec((_D, fco), lambda i: (0, 0)),
            pl.BlockSpec((1, fco), lambda i: (0, 0)),
        ],
        out_specs=pl.BlockSpec((_B, fco), lambda i: (0, 0)),
        out_shape=jax.ShapeDtypeStruct((_B, fco), jnp.float32),
        scratch_shapes=[
            pltpu.VMEM((_B, _D), jnp.float32),
            pltpu.VMEM((_B, _D), jnp.float32),
        ],
    )(acc, g, deg_parts, b2r, batch3, wfc, bfcr)


def kernel(x, edge_index, batch, W1, b1, W2, b2, Wfc, bfc):
    pad = _EP - _E
    lanes = jnp.arange(pad, dtype=jnp.int32) % 128
    src = jnp.concatenate([edge_index[0], lanes])
    dst = jnp.concatenate([edge_index[1], _N + lanes])
    src2 = src.reshape(_EP // 128, 128)
    dst2 = dst.reshape(_EP // 128, 128)

    deg_parts = _deg_kernel(dst).reshape(_NW, _HN)
    xp = jnp.pad(x, ((0, _NPAD - _N), (0, 0)))
    batchp = jnp.pad(batch, (0, _NPAD - _N), constant_values=_B)

    g1 = _tc1(xp, W1, deg_parts)
    acc1 = _agg_kernel(g1, src2, dst2)
    g2 = _tc2(acc1, g1, deg_parts, b1.reshape(1, _D), W2)
    acc2 = _agg_kernel(g2, src2, dst2)
    out = _tc3(acc2, g2, deg_parts, b2.reshape(1, _D), batchp.reshape(_G, 1, _R),
               Wfc, bfc.reshape(1, -1))
    return out.reshape(_B, 4, 32, 64)


# speedup vs baseline: 1.0505x; 1.0170x over previous
import functools

import jax
import jax.numpy as jnp
from jax import lax
from jax.experimental import pallas as pl
from jax.experimental.pallas import tpu as pltpu
from jax.experimental.pallas import tpu_sc as plsc

_NC, _NS, _L = 2, 16, 16
_NW = _NC * _NS
_N = 10000
_D = 128
_E = 320000
_B = 16
_EP = 327680
_CH = _EP // _NW
_RW = _CH // 128
_CHKR = 16
_NCHK = _RW // _CHKR
_NPAD = 10240
_RPT = _NPAD // _NS
_HR = 80
_HN = _HR * 128

_R = 5120
_G = _NPAD // _R


def _mesh():
    return plsc.VectorSubcoreMesh(core_axis_name="c", subcore_axis_name="s",
                                  num_cores=_NC, num_subcores=_NS)


@functools.partial(
    pl.kernel,
    out_type=jax.ShapeDtypeStruct((_NW * _HN,), jnp.float32),
    mesh=_mesh(),
    compiler_params=pltpu.CompilerParams(needs_layout_passes=False),
    scratch_types=[
        pltpu.VMEM((_CH,), jnp.int32),
        pltpu.VMEM((_HN,), jnp.float32),
    ])
def _deg_kernel(dst_hbm, out_hbm, dst_v, hist_v):
    cid = lax.axis_index("c")
    sid = lax.axis_index("s")
    w = cid * _NS + sid
    zeros = jnp.zeros((_L,), jnp.float32)
    ones = jnp.ones((_L,), jnp.float32)

    def zbody(i, carry):
        hist_v[pl.ds(i * _L, _L)] = zeros
        return carry
    lax.fori_loop(0, _HN // _L, zbody, 0)

    pltpu.sync_copy(dst_hbm.at[pl.ds(w * _CH, _CH)], dst_v)

    def body(i, carry):
        v = dst_v[pl.ds(i * _L, _L)]
        plsc.addupdate_scatter(hist_v, [v], ones)
        return carry
    lax.fori_loop(0, _CH // _L, body, 0)

    pltpu.sync_copy(hist_v, out_hbm.at[pl.ds(w * _HN, _HN)])


@functools.partial(
    pl.kernel,
    out_type=jax.ShapeDtypeStruct((_NC, _NPAD, _D), jnp.float32),
    mesh=_mesh(),
    compiler_params=pltpu.CompilerParams(needs_layout_passes=False),
    scratch_types=[
        [pltpu.VMEM((_CHKR, 128), jnp.int32)] * 2,
        [pltpu.VMEM((_CHKR, 128), jnp.int32)] * 2,
        [pltpu.VMEM((128, _D), jnp.float32)] * 2,
        [pltpu.SemaphoreType.DMA] * 2,
        [pltpu.SemaphoreType.DMA] * 2,
        [pltpu.SemaphoreType.DMA] * 2,
        pltpu.VMEM_SHARED((_NPAD, _D), jnp.float32),
    ])
def _agg_kernel(g_hbm, src_hbm, dst_hbm, out_hbm,
                src_c, dst_c, gbufs, gsems, ssems, dsems, acc_sh):
    cid = lax.axis_index("c")
    sid = lax.axis_index("s")
    w = cid * _NS + sid
    base = w * _RW

    def load_chunk(c):
        p = c % 2
        off = pl.multiple_of(base + c * _CHKR, 8)
        pltpu.async_copy(src_hbm.at[pl.ds(off, _CHKR)], src_c[p], ssems[p])
        pltpu.async_copy(dst_hbm.at[pl.ds(off, _CHKR)], dst_c[p], dsems[p])

    def wait_chunk(c):
        p = c % 2
        off = pl.multiple_of(base + c * _CHKR, 8)
        pltpu.make_async_copy(src_hbm.at[pl.ds(off, _CHKR)], src_c[p], ssems[p]).wait()
        pltpu.make_async_copy(dst_hbm.at[pl.ds(off, _CHKR)], dst_c[p], dsems[p]).wait()

    def issue_gather(p, row, b):
        pltpu.async_copy(g_hbm.at[src_c[p].at[row]], gbufs[b], gsems[b])

    def wait_gather(p, row, b):
        pltpu.make_async_copy(g_hbm.at[src_c[p].at[row]], gbufs[b], gsems[b]).wait()

    def scatter(p, row, b):
        pltpu.sync_copy(gbufs[b], acc_sh.at[dst_c[p].at[row]], add=True)

    ioff = pl.multiple_of(sid * _RPT, 8)
    pltpu.sync_copy(g_hbm.at[pl.ds(ioff, _RPT)], acc_sh.at[pl.ds(ioff, _RPT)])

    load_chunk(0)
    wait_chunk(0)
    load_chunk(1)
    plsc.subcore_barrier()

    for b in range(2):
        issue_gather(0, b, b)

    for c in range(_NCHK):
        p = c % 2

        def grp_body(grp, carry, p=p):
            for b in range(2):
                j = grp * 2 + b
                wait_gather(p, j, b)
                scatter(p, j, b)
                issue_gather(p, j + 2, b)
            return carry
        lax.fori_loop(0, _CHKR // 2 - 1, grp_body, 0)

        for b in range(2):
            j = _CHKR - 2 + b
            wait_gather(p, j, b)
            scatter(p, j, b)

        if c < _NCHK - 1:
            pn = (c + 1) % 2
            wait_chunk(c + 1)
            if c + 2 < _NCHK:
                load_chunk(c + 2)
            for b in range(2):
                issue_gather(pn, b, b)

    plsc.subcore_barrier()
    pltpu.sync_copy(acc_sh.at[pl.ds(ioff, _RPT)],
                    out_hbm.at[cid, pl.ds(ioff, _RPT)])


def _dis_col(degp_ref):
    ones = jnp.ones((_NW, 1), jnp.float32)
    deg = lax.dot_general(degp_ref[...], ones, (((0,), (0,)), ((), ())),
                          preferred_element_type=jnp.float32) + 1.0
    return lax.rsqrt(deg)


def _tc1_body(x_ref, w_ref, degp_ref, out_ref):
    dis = _dis_col(degp_ref)
    rowid = lax.broadcasted_iota(jnp.int32, (_R, 1), 0) + pl.program_id(0) * _R
    dis = jnp.where(rowid < _N, dis, 0.0)
    out_ref[...] = jnp.dot(x_ref[...], w_ref[...],
                           preferred_element_type=jnp.float32) * dis


def _tc1(x, w1, deg_parts):
    return pl.pallas_call(
        _tc1_body,
        grid=(_G,),
        in_specs=[
            pl.BlockSpec((_R, _D), lambda i: (i, 0)),
            pl.BlockSpec((_D, _D), lambda i: (0, 0)),
            pl.BlockSpec((_NW, _R), lambda i: (0, i)),
        ],
        out_specs=pl.BlockSpec((_R, _D), lambda i: (i, 0)),
        out_shape=jax.ShapeDtypeStruct((_NPAD, _D), jnp.float32),
    )(x, w1, deg_parts)


def _tc2_body(acc_ref, g_ref, degp_ref, b_ref, w_ref, out_ref):
    dis = _dis_col(degp_ref)
    a = acc_ref[0] + acc_ref[1] - g_ref[...]
    h = jnp.maximum(a * dis + b_ref[...], 0.0)
    out_ref[...] = jnp.dot(h, w_ref[...],
                           preferred_element_type=jnp.float32) * dis


def _tc2(acc, g, deg_parts, b1r, w2):
    return pl.pallas_call(
        _tc2_body,
        grid=(_G,),
        in_specs=[
            pl.BlockSpec((_NC, _R, _D), lambda i: (0, i, 0)),
            pl.BlockSpec((_R, _D), lambda i: (i, 0)),
            pl.BlockSpec((_NW, _R), lambda i: (0, i)),
            pl.BlockSpec((1, _D), lambda i: (0, 0)),
            pl.BlockSpec((_D, _D), lambda i: (0, 0)),
        ],
        out_specs=pl.BlockSpec((_R, _D), lambda i: (i, 0)),
        out_shape=jax.ShapeDtypeStruct((_NPAD, _D), jnp.float32),
    )(acc, g, deg_parts, b1r, w2)


def _tc3_body(acc_ref, g_ref, degp_ref, b_ref, batch_ref, wfc_ref, bfc_ref,
              out_ref, psum, cnt):
    i = pl.program_id(0)

    @pl.when(i == 0)
    def _():
        psum[...] = jnp.zeros_like(psum)
        cnt[...] = jnp.zeros_like(cnt)

    dis = _dis_col(degp_ref)
    a = acc_ref[0] + acc_ref[1] - g_ref[...]
    h = jnp.maximum(a * dis + b_ref[...], 0.0)
    ids = lax.broadcasted_iota(jnp.int32, (_B, _R), 0)
    mask = (ids == batch_ref[...].reshape(1, _R)).astype(jnp.float32)
    psum[...] += jnp.dot(mask, h, preferred_element_type=jnp.float32)
    cnt[...] += jnp.broadcast_to(jnp.sum(mask, axis=1, keepdims=True), (_B, _D))

    @pl.when(i == _G - 1)
    def _():
        pooled = psum[...] / jnp.maximum(cnt[...], 1.0)
        out_ref[...] = jnp.dot(pooled, wfc_ref[...],
                               preferred_element_type=jnp.float32) + bfc_ref[...]


def _tc3(acc, g, deg_parts, b2r, batch3, wfc, bfcr):
    fco = wfc.shape[1]
    return pl.pallas_call(
        _tc3_body,
        grid=(_G,),
        in_specs=[
            pl.BlockSpec((_NC, _R, _D), lambda i: (0, i, 0)),
            pl.BlockSpec((_R, _D), lambda i: (i, 0)),
            pl.BlockSpec((_NW, _R), lambda i: (0, i)),
            pl.BlockSpec((1, _D), lambda i: (0, 0)),
            pl.BlockSpec((1, 1, _R), lambda i: (i, 0, 0)),
            pl.BlockSpec((_D, fco), lambda i: (0, 0)),
            pl.BlockSpec((1, fco), lambda i: (0, 0)),
        ],
        out_specs=pl.BlockSpec((_B, fco), lambda i: (0, 0)),
        out_shape=jax.ShapeDtypeStruct((_B, fco), jnp.float32),
        scratch_shapes=[
            pltpu.VMEM((_B, _D), jnp.float32),
            pltpu.VMEM((_B, _D), jnp.float32),
        ],
    )(acc, g, deg_parts, b2r, batch3, wfc, bfcr)


def kernel(x, edge_index, batch, W1, b1, W2, b2, Wfc, bfc):
    pad = _EP - _E
    lanes = jnp.arange(pad, dtype=jnp.int32) % 128
    src = jnp.concatenate([edge_index[0], lanes])
    dst = jnp.concatenate([edge_index[1], _N + lanes])
    src2 = src.reshape(_EP // 128, 128)
    dst2 = dst.reshape(_EP // 128, 128)

    deg_parts = _deg_kernel(dst).reshape(_NW, _HN)
    xp = jnp.pad(x, ((0, _NPAD - _N), (0, 0)))
    batchp = jnp.pad(batch, (0, _NPAD - _N), constant_values=_B)

    g1 = _tc1(xp, W1, deg_parts)
    acc1 = _agg_kernel(g1, src2, dst2)
    g2 = _tc2(acc1, g1, deg_parts, b1.reshape(1, _D), W2)
    acc2 = _agg_kernel(g2, src2, dst2)
    out = _tc3(acc2, g2, deg_parts, b2.reshape(1, _D), batchp.reshape(_G, 1, _R),
               Wfc, bfc.reshape(1, -1))
    return out.reshape(_B, 4, 32, 64)
